# baseline (speedup 1.0000x reference)
import jax
import jax.numpy as jnp
from jax.experimental import pallas as pl

_BLOCK = 16672


def _body(x_ref, t_ref, pred_ref, datan_ref, taskn_ref):
    t = t_ref[...]
    tn = t / jnp.maximum(
        jnp.sqrt(jnp.sum(t * t, axis=1, keepdims=True)), 1e-12)

    @pl.when(pl.program_id(0) == 0)
    def _():
        taskn_ref[...] = tn

    x = x_ref[...]
    xn = x / jnp.maximum(
        jnp.sqrt(jnp.sum(x * x, axis=1, keepdims=True)), 1e-12)
    datan_ref[...] = xn
    pred_ref[...] = jax.lax.dot_general(
        xn, tn,
        dimension_numbers=(((1,), (1,)), ((), ())),
        preferred_element_type=jnp.float32)


def kernel(graph_feature, graph_targets_id_batch, graph_targets_id,
           graph_targets_value, task_emb):
    n, d = graph_feature.shape
    k = task_emb.shape[0]
    blk = min(_BLOCK, n)
    pred, data_n, task_n = pl.pallas_call(
        _body,
        grid=(pl.cdiv(n, blk),),
        in_specs=[
            pl.BlockSpec((blk, d), lambda i: (i, 0)),
            pl.BlockSpec((k, d), lambda i: (0, 0)),
        ],
        out_specs=[
            pl.BlockSpec((blk, k), lambda i: (i, 0)),
            pl.BlockSpec((blk, d), lambda i: (i, 0)),
            pl.BlockSpec((k, d), lambda i: (0, 0)),
        ],
        out_shape=[
            jax.ShapeDtypeStruct((n, k), jnp.float32),
            jax.ShapeDtypeStruct((n, d), jnp.float32),
            jax.ShapeDtypeStruct((k, d), jnp.float32),
        ],
    )(graph_feature, task_emb)
    return (pred, data_n, task_n)


# speedup vs baseline: 1.0058x; 1.0058x over previous
import jax
import jax.numpy as jnp
from jax.experimental import pallas as pl
from jax.experimental.pallas import tpu as pltpu

_N_CHUNKS = 5


def _l2n(x):
    return x / jnp.maximum(
        jnp.sqrt(jnp.sum(x * x, axis=1, keepdims=True)), 1e-12)


def _make_body(n, blk, sizes):
    offs = [sum(sizes[:i]) for i in range(len(sizes))]
    nc = len(sizes)

    def _body(x_hbm, t_hbm, pred_hbm, datan_hbm, taskn_hbm,
              xb, pb, tb, tnb, in_sem, dn_sem, pr_sem, t_sem, tn_sem):
        def in_cp(i):
            return pltpu.make_async_copy(
                x_hbm.at[pl.ds(offs[i], sizes[i])],
                xb.at[i % 3, pl.ds(0, sizes[i])], in_sem.at[i % 3])

        def dn_cp(i):
            return pltpu.make_async_copy(
                xb.at[i % 3, pl.ds(0, sizes[i])],
                datan_hbm.at[pl.ds(offs[i], sizes[i])], dn_sem.at[i % 3])

        def pr_cp(i):
            return pltpu.make_async_copy(
                pb.at[i % 2, pl.ds(0, sizes[i])],
                pred_hbm.at[pl.ds(offs[i], sizes[i])], pr_sem.at[i % 2])

        t_cp = pltpu.make_async_copy(t_hbm, tb, t_sem)
        tn_cp = pltpu.make_async_copy(tnb, taskn_hbm, tn_sem)

        t_cp.start()
        in_cp(0).start()
        t_cp.wait()
        tn = _l2n(tb[...])
        tnb[...] = tn
        tn_cp.start()

        for i in range(nc):
            if i + 1 < nc:
                if i + 1 >= 3:
                    dn_cp(i - 2).wait()
                in_cp(i + 1).start()
            in_cp(i).wait()
            if i >= 2:
                pr_cp(i - 2).wait()
            x = xb[i % 3, pl.ds(0, sizes[i]), :]
            xn = _l2n(x)
            xb[i % 3, pl.ds(0, sizes[i]), :] = xn
            pb[i % 2, pl.ds(0, sizes[i]), :] = jax.lax.dot_general(
                xn, tn,
                dimension_numbers=(((1,), (1,)), ((), ())),
                preferred_element_type=jnp.float32)
            dn_cp(i).start()
            pr_cp(i).start()

        for i in range(max(0, nc - 3), nc):
            dn_cp(i).wait()
        for i in range(max(0, nc - 2), nc):
            pr_cp(i).wait()
        tn_cp.wait()

    return _body


def kernel(graph_feature, graph_targets_id_batch, graph_targets_id,
           graph_targets_value, task_emb):
    n, d = graph_feature.shape
    k = task_emb.shape[0]
    blk = ((n + _N_CHUNKS - 1) // _N_CHUNKS + 7) // 8 * 8
    sizes = []
    left = n
    while left > 0:
        s = min(blk, left)
        sizes.append(s)
        left -= s
    pred, data_n, task_n = pl.pallas_call(
        _make_body(n, blk, sizes),
        in_specs=[
            pl.BlockSpec(memory_space=pl.ANY),
            pl.BlockSpec(memory_space=pl.ANY),
        ],
        out_specs=[
            pl.BlockSpec(memory_space=pl.ANY),
            pl.BlockSpec(memory_space=pl.ANY),
            pl.BlockSpec(memory_space=pl.ANY),
        ],
        out_shape=[
            jax.ShapeDtypeStruct((n, k), jnp.float32),
            jax.ShapeDtypeStruct((n, d), jnp.float32),
            jax.ShapeDtypeStruct((k, d), jnp.float32),
        ],
        scratch_shapes=[
            pltpu.VMEM((3, blk, d), jnp.float32),
            pltpu.VMEM((2, blk, k), jnp.float32),
            pltpu.VMEM((k, d), jnp.float32),
            pltpu.VMEM((k, d), jnp.float32),
            pltpu.SemaphoreType.DMA((3,)),
            pltpu.SemaphoreType.DMA((3,)),
            pltpu.SemaphoreType.DMA((2,)),
            pltpu.SemaphoreType.DMA,
            pltpu.SemaphoreType.DMA,
        ],
    )(graph_feature, task_emb)
    return (pred, data_n, task_n)


# speedup vs baseline: 1.0545x; 1.0484x over previous
import jax
import jax.numpy as jnp
from jax.experimental import pallas as pl
from jax.experimental.pallas import tpu as pltpu

_N_CHUNKS = 5


def _l2n(x):
    return x / jnp.maximum(
        jnp.sqrt(jnp.sum(x * x, axis=1, keepdims=True)), 1e-12)


def _make_body(n, blk, sizes):
    offs = [sum(sizes[:i]) for i in range(len(sizes))]
    nc = len(sizes)

    def _body(x_hbm, t_hbm, pred_hbm, datan_hbm, taskn_hbm,
              xb, pb, tb, tnb, in_sem, dn_sem, pr_sem, t_sem, tn_sem):
        def in_cp(i):
            return pltpu.make_async_copy(
                x_hbm.at[pl.ds(offs[i], sizes[i])],
                xb.at[i % 3, pl.ds(0, sizes[i])], in_sem.at[i % 3])

        def dn_cp(i):
            return pltpu.make_async_copy(
                xb.at[i % 3, pl.ds(0, sizes[i])],
                datan_hbm.at[pl.ds(offs[i], sizes[i])], dn_sem.at[i % 3])

        def pr_cp(i):
            return pltpu.make_async_copy(
                pb.at[i % 2, pl.ds(0, sizes[i])],
                pred_hbm.at[pl.ds(offs[i], sizes[i])], pr_sem.at[i % 2])

        t_cp = pltpu.make_async_copy(t_hbm, tb, t_sem)
        tn_cp = pltpu.make_async_copy(tnb, taskn_hbm, tn_sem)

        t_cp.start()
        in_cp(0).start()
        t_cp.wait()
        tn = _l2n(tb[...])
        tnb[...] = tn
        tn_cp.start()

        for i in range(nc):
            if i + 1 < nc:
                if i + 1 >= 3:
                    dn_cp(i - 2).wait()
                in_cp(i + 1).start()
            in_cp(i).wait()
            if i >= 2:
                pr_cp(i - 2).wait()
            x = xb[i % 3, pl.ds(0, sizes[i]), :]
            xn = _l2n(x)
            xb[i % 3, pl.ds(0, sizes[i]), :] = xn
            dn_cp(i).start()
            pb[i % 2, pl.ds(0, sizes[i]), :] = jax.lax.dot_general(
                xn, tn,
                dimension_numbers=(((1,), (1,)), ((), ())),
                preferred_element_type=jnp.float32)
            pr_cp(i).start()

        for i in range(max(0, nc - 3), nc):
            dn_cp(i).wait()
        for i in range(max(0, nc - 2), nc):
            pr_cp(i).wait()
        tn_cp.wait()

    return _body


def kernel(graph_feature, graph_targets_id_batch, graph_targets_id,
           graph_targets_value, task_emb):
    n, d = graph_feature.shape
    k = task_emb.shape[0]
    blk = ((n + _N_CHUNKS - 1) // _N_CHUNKS + 7) // 8 * 8
    first = min((blk * 2 // 5 + 7) // 8 * 8, n)
    sizes = [first]
    left = n - first
    while left > blk:
        sizes.append(blk)
        left -= blk
    if left > 0:
        sizes.append(left)
    pred, data_n, task_n = pl.pallas_call(
        _make_body(n, blk, sizes),
        in_specs=[
            pl.BlockSpec(memory_space=pl.ANY),
            pl.BlockSpec(memory_space=pl.ANY),
        ],
        out_specs=[
            pl.BlockSpec(memory_space=pl.ANY),
            pl.BlockSpec(memory_space=pl.ANY),
            pl.BlockSpec(memory_space=pl.ANY),
        ],
        out_shape=[
            jax.ShapeDtypeStruct((n, k), jnp.float32),
            jax.ShapeDtypeStruct((n, d), jnp.float32),
            jax.ShapeDtypeStruct((k, d), jnp.float32),
        ],
        scratch_shapes=[
            pltpu.VMEM((3, blk, d), jnp.float32),
            pltpu.VMEM((2, blk, k), jnp.float32),
            pltpu.VMEM((k, d), jnp.float32),
            pltpu.VMEM((k, d), jnp.float32),
            pltpu.SemaphoreType.DMA((3,)),
            pltpu.SemaphoreType.DMA((3,)),
            pltpu.SemaphoreType.DMA((2,)),
            pltpu.SemaphoreType.DMA,
            pltpu.SemaphoreType.DMA,
        ],
    )(graph_feature, task_emb)
    return (pred, data_n, task_n)
